# baseline (speedup 1.0000x reference)
import functools

import jax
import jax.numpy as jnp
from jax import lax
from jax.experimental import pallas as pl
from jax.experimental.pallas import tpu as pltpu
from jax.experimental.pallas import tpu_sc as plsc

DIM = 300
BATCH = 16384
MARGIN = 0.5
EPS = 1e-12

NC, NS, L = 2, 16, 16
NW = NC * NS
RPW = BATCH // NW
C = 16
NCHUNK = RPW // C
GPC = C // L
U = 4
SLEN = (128, 128, DIM - 256)


def _newton_rsqrt(x):
    i = plsc.bitcast(x, jnp.int32)
    y = plsc.bitcast(jnp.int32(0x5F3759DF) - (i >> 1), jnp.float32)
    for _ in range(3):
        y = y * (1.5 - 0.5 * x * y * y)
    return y


_mesh = plsc.VectorSubcoreMesh(core_axis_name="c", subcore_axis_name="s")


@functools.partial(
    pl.kernel,
    out_type=jax.ShapeDtypeStruct((NW, L), jnp.float32),
    mesh=_mesh,
    compiler_params=pltpu.CompilerParams(use_tc_tiling_on_sc=True,
                                         needs_layout_passes=False),
    scratch_types=(
        [pltpu.VMEM((RPW,), jnp.int32) for _ in range(5)]
        + [pltpu.VMEM((C, 128), jnp.float32) for _ in range(30)]
        + [pltpu.VMEM((L,), jnp.float32)]
        + [pltpu.SemaphoreType.DMA, pltpu.SemaphoreType.DMA]
    ),
)
def _joie_sc(h_idx_hbm, r_idx_hbm, t_idx_hbm, hn_idx_hbm, tn_idx_hbm,
             ht_hbm, r_hbm, out_hbm,
             hi_v, ri_v, ti_v, hni_v, tni_v,
             *rest):
    bufs = rest[:30]
    loss_v = rest[30]
    sems = rest[31:33]

    wid = lax.axis_index("s") * NC + lax.axis_index("c")
    base = wid * RPW

    idx_refs = (hi_v, ti_v, hni_v, tni_v, ri_v)
    for ihbm, iv in ((h_idx_hbm, hi_v), (t_idx_hbm, ti_v),
                     (hn_idx_hbm, hni_v), (tn_idx_hbm, tni_v),
                     (r_idx_hbm, ri_v)):
        pltpu.sync_copy(ihbm.at[pl.ds(base, RPW)], iv)

    tables = (ht_hbm, ht_hbm, ht_hbm, ht_hbm, r_hbm)

    def bank_bufs(b):
        return [[bufs[b * 15 + role * 3 + s] for s in range(3)]
                for role in range(5)]

    def dma_descs(g, b):
        bb = bank_bufs(b)
        descs = []
        for role in range(5):
            iv = idx_refs[role].at[pl.ds(g * C, C)]
            for s in range(3):
                cstart = jnp.int32(s * 128) + wid * 0
                descs.append(pltpu.make_async_copy(
                    tables[role].at[iv, pl.ds(cstart, 128)],
                    bb[role][s], sems[b]))
        return descs

    def start(g, b):
        for d in dma_descs(g, b):
            d.start()

    def wait(g, b):
        for d in dma_descs(g, b):
            d.wait()

    zero = jnp.zeros((L,), jnp.float32)

    lane = lax.iota(jnp.int32, L)
    tailmask = (lane < SLEN[2] - 32).astype(jnp.float32)

    def compute(b, loss_acc):
        bb = bank_bufs(b)
        for grp in range(GPC):

            def row_body(r, carry):
                vsp, vsn, vnh, vnt, vnhn, vntn = carry
                rr = r + grp * L
                sp = sn = nh = nt = nhn = ntn = zero
                for s in range(3):
                    for c in range(0, SLEN[s] + 15 & ~15, L):
                        hv = bb[0][s][rr, pl.ds(c, L)]
                        tv = bb[1][s][rr, pl.ds(c, L)]
                        hnv = bb[2][s][rr, pl.ds(c, L)]
                        tnv = bb[3][s][rr, pl.ds(c, L)]
                        rv = bb[4][s][rr, pl.ds(c, L)]
                        if s == 2 and c + L > SLEN[s]:
                            hv = hv * tailmask
                            tv = tv * tailmask
                            hnv = hnv * tailmask
                            tnv = tnv * tailmask
                        sp = sp + rv * hv * tv
                        sn = sn + rv * hnv * tnv
                        nh = nh + hv * hv
                        nt = nt + tv * tv
                        nhn = nhn + hnv * hnv
                        ntn = ntn + tnv * tnv
                sel = lane == r
                vsp = jnp.where(sel, jnp.sum(sp), vsp)
                vsn = jnp.where(sel, jnp.sum(sn), vsn)
                vnh = jnp.where(sel, jnp.sum(nh), vnh)
                vnt = jnp.where(sel, jnp.sum(nt), vnt)
                vnhn = jnp.where(sel, jnp.sum(nhn), vnhn)
                vntn = jnp.where(sel, jnp.sum(ntn), vntn)
                return vsp, vsn, vnh, vnt, vnhn, vntn

            carry = (zero,) * 6
            for r in range(L):
                carry = row_body(r, carry)
            sp, sn, nh, nt, nhn, ntn = carry
            inv_h = 1.0 / jnp.maximum(nh * _newton_rsqrt(nh), EPS)
            inv_t = 1.0 / jnp.maximum(nt * _newton_rsqrt(nt), EPS)
            inv_hn = 1.0 / jnp.maximum(nhn * _newton_rsqrt(nhn), EPS)
            inv_tn = 1.0 / jnp.maximum(ntn * _newton_rsqrt(ntn), EPS)
            pos = sp * inv_h * inv_t
            neg = sn * inv_hn * inv_tn
            loss_acc = loss_acc + jnp.maximum(neg - pos + MARGIN, 0.0)
        return loss_acc

    start(0, 0)
    start(1, 1)

    def outer(i, loss_acc):
        for b in range(2):
            g = i * 2 + b
            wait(g, b)
            loss_acc = compute(b, loss_acc)

            @pl.when(g + 2 < NCHUNK)
            def _():
                start(g + 2, b)
        return loss_acc

    loss = lax.fori_loop(0, NCHUNK // 2, outer, zero)
    loss_v[...] = loss
    pltpu.sync_copy(loss_v, out_hbm.at[wid])


TRB = 8192


def _tr_body(i_ref, o_ref):
    o_ref[...] = i_ref[...].T


def _transpose_tc(u):
    n = u.shape[1]
    return pl.pallas_call(
        _tr_body,
        grid=(pl.cdiv(n, TRB),),
        in_specs=[pl.BlockSpec((DIM, TRB), lambda i: (0, i))],
        out_specs=pl.BlockSpec((TRB, DIM), lambda i: (i, 0)),
        out_shape=jax.ShapeDtypeStruct((n, DIM), jnp.float32),
        compiler_params=pltpu.CompilerParams(
            dimension_semantics=("parallel",)),
    )(u)


def kernel(A_h_index, A_r_index, A_t_index, A_hn_index, A_tn_index, ht1, r1):
    tt = _transpose_tc(ht1.T)
    partials = _joie_sc(A_h_index.astype(jnp.int32),
                        A_r_index.astype(jnp.int32),
                        A_t_index.astype(jnp.int32),
                        A_hn_index.astype(jnp.int32),
                        A_tn_index.astype(jnp.int32),
                        tt, r1)
    return jnp.sum(partials) / BATCH


# speedup vs baseline: 1.1060x; 1.1060x over previous
import functools

import jax
import jax.numpy as jnp
from jax import lax
from jax.experimental import pallas as pl
from jax.experimental.pallas import tpu as pltpu
from jax.experimental.pallas import tpu_sc as plsc

DIM = 300
BATCH = 16384
MARGIN = 0.5
EPS = 1e-12

NC, NS, L = 2, 16, 16
NW = NC * NS
RPW = BATCH // NW
C = 16
NCHUNK = RPW // C
GPC = C // L
U = 4
SLEN = (128, 128, DIM - 256)


def _newton_rsqrt(x):
    i = plsc.bitcast(x, jnp.int32)
    y = plsc.bitcast(jnp.int32(0x5F3759DF) - (i >> 1), jnp.float32)
    for _ in range(3):
        y = y * (1.5 - 0.5 * x * y * y)
    return y


_mesh = plsc.VectorSubcoreMesh(core_axis_name="c", subcore_axis_name="s")


@functools.partial(
    pl.kernel,
    out_type=jax.ShapeDtypeStruct((NW, L), jnp.float32),
    mesh=_mesh,
    compiler_params=pltpu.CompilerParams(use_tc_tiling_on_sc=True,
                                         needs_layout_passes=False),
    scratch_types=(
        [pltpu.VMEM((RPW,), jnp.int32) for _ in range(5)]
        + [pltpu.VMEM((RPW,), jnp.int32) for _ in range(4)]
        + [pltpu.VMEM((C, 128), jnp.float32) for _ in range(30)]
        + [pltpu.VMEM((L,), jnp.float32)]
        + [pltpu.SemaphoreType.DMA, pltpu.SemaphoreType.DMA]
    ),
)
def _joie_sc(h_idx_hbm, r_idx_hbm, t_idx_hbm, hn_idx_hbm, tn_idx_hbm,
             ht_hbm, httail_hbm, r_hbm, out_hbm,
             hi_v, ri_v, ti_v, hni_v, tni_v,
             *rest):
    jv_refs = rest[:4]
    bufs = rest[4:34]
    loss_v = rest[34]
    sems = rest[35:37]

    wid = lax.axis_index("s") * NC + lax.axis_index("c")
    base = wid * RPW

    idx_refs = (hi_v, ti_v, hni_v, tni_v, ri_v)
    for ihbm, iv in ((h_idx_hbm, hi_v), (t_idx_hbm, ti_v),
                     (hn_idx_hbm, hni_v), (tn_idx_hbm, tni_v),
                     (r_idx_hbm, ri_v)):
        pltpu.sync_copy(ihbm.at[pl.ds(base, RPW)], iv)
    for role in range(4):
        for gg in range(RPW // L):
            v = idx_refs[role][pl.ds(gg * L, L)]
            jv_refs[role][pl.ds(gg * L, L)] = (
                ((v >> 13) << 12) + (v & 4095))

    tables = (ht_hbm, ht_hbm, ht_hbm, ht_hbm, r_hbm)

    def bank_bufs(b):
        return [[bufs[b * 15 + role * 3 + s] for s in range(3)]
                for role in range(5)]

    def dma_descs(g, b):
        bb = bank_bufs(b)
        descs = []
        for role in range(5):
            iv = idx_refs[role].at[pl.ds(g * C, C)]
            if role < 4:
                jv = jv_refs[role].at[pl.ds(g * C, C)]
                descs.append(pltpu.make_async_copy(
                    tables[role].at[iv, pl.ds(0, 128)], bb[role][0], sems[b]))
                descs.append(pltpu.make_async_copy(
                    tables[role].at[iv, pl.ds(128, 128)], bb[role][1], sems[b]))
                descs.append(pltpu.make_async_copy(
                    httail_hbm.at[jv], bb[role][2], sems[b]))
            else:
                for s in range(3):
                    cstart = jnp.int32(s * 128) + wid * 0
                    descs.append(pltpu.make_async_copy(
                        tables[role].at[iv, pl.ds(cstart, 128)],
                        bb[role][s], sems[b]))
        return descs

    def start(g, b):
        for d in dma_descs(g, b):
            d.start()

    def wait(g, b):
        for d in dma_descs(g, b):
            d.wait()

    zero = jnp.zeros((L,), jnp.float32)

    lane = lax.iota(jnp.int32, L)
    tailmask = (lane < SLEN[2] - 32).astype(jnp.float32)

    def compute(g, b, loss_acc):
        bb = bank_bufs(b)
        for grp in range(GPC):
            pov = [((idx_refs[role][pl.ds(g * C + grp * L, L)] >> 12) & 1)
                   * 64 for role in range(4)]

            def row_body(r, carry):
                vsp, vsn, vnh, vnt, vnhn, vntn = carry
                rr = r + grp * L
                po = [pov[role][r] for role in range(4)]
                sp = sn = nh = nt = nhn = ntn = zero
                for s in range(3):
                    for c in range(0, SLEN[s] + 15 & ~15, L):
                        if s == 2:
                            hv = bb[0][s][rr, pl.ds(po[0] + c, L)]
                            tv = bb[1][s][rr, pl.ds(po[1] + c, L)]
                            hnv = bb[2][s][rr, pl.ds(po[2] + c, L)]
                            tnv = bb[3][s][rr, pl.ds(po[3] + c, L)]
                        else:
                            hv = bb[0][s][rr, pl.ds(c, L)]
                            tv = bb[1][s][rr, pl.ds(c, L)]
                            hnv = bb[2][s][rr, pl.ds(c, L)]
                            tnv = bb[3][s][rr, pl.ds(c, L)]
                        rv = bb[4][s][rr, pl.ds(c, L)]
                        if s == 2 and c + L > SLEN[s]:
                            hv = hv * tailmask
                            tv = tv * tailmask
                            hnv = hnv * tailmask
                            tnv = tnv * tailmask
                        sp = sp + rv * hv * tv
                        sn = sn + rv * hnv * tnv
                        nh = nh + hv * hv
                        nt = nt + tv * tv
                        nhn = nhn + hnv * hnv
                        ntn = ntn + tnv * tnv
                sel = lane == r
                vsp = jnp.where(sel, jnp.sum(sp), vsp)
                vsn = jnp.where(sel, jnp.sum(sn), vsn)
                vnh = jnp.where(sel, jnp.sum(nh), vnh)
                vnt = jnp.where(sel, jnp.sum(nt), vnt)
                vnhn = jnp.where(sel, jnp.sum(nhn), vnhn)
                vntn = jnp.where(sel, jnp.sum(ntn), vntn)
                return vsp, vsn, vnh, vnt, vnhn, vntn

            carry = (zero,) * 6
            for r in range(L):
                carry = row_body(r, carry)
            sp, sn, nh, nt, nhn, ntn = carry
            inv_h = 1.0 / jnp.maximum(nh * _newton_rsqrt(nh), EPS)
            inv_t = 1.0 / jnp.maximum(nt * _newton_rsqrt(nt), EPS)
            inv_hn = 1.0 / jnp.maximum(nhn * _newton_rsqrt(nhn), EPS)
            inv_tn = 1.0 / jnp.maximum(ntn * _newton_rsqrt(ntn), EPS)
            pos = sp * inv_h * inv_t
            neg = sn * inv_hn * inv_tn
            loss_acc = loss_acc + jnp.maximum(neg - pos + MARGIN, 0.0)
        return loss_acc

    start(0, 0)
    start(1, 1)

    def outer(i, loss_acc):
        for b in range(2):
            g = i * 2 + b
            wait(g, b)
            loss_acc = compute(g, b, loss_acc)

            @pl.when(g + 2 < NCHUNK)
            def _():
                start(g + 2, b)
        return loss_acc

    loss = lax.fori_loop(0, NCHUNK // 2, outer, zero)
    loss_v[...] = loss
    pltpu.sync_copy(loss_v, out_hbm.at[wid])


TRB = 8192


def _tr_body(i_ref, om_ref, ot_ref):
    x = i_ref[...]
    om_ref[...] = x[:256].T
    t64 = jnp.pad(x[256:].T, ((0, 0), (0, 64 - (DIM - 256))))
    ot_ref[...] = jnp.concatenate([t64[:TRB // 2], t64[TRB // 2:]], axis=1)


def _transpose_tc(u):
    n = u.shape[1]
    return pl.pallas_call(
        _tr_body,
        grid=(pl.cdiv(n, TRB),),
        in_specs=[pl.BlockSpec((DIM, TRB), lambda i: (0, i))],
        out_specs=[pl.BlockSpec((TRB, 256), lambda i: (i, 0)),
                   pl.BlockSpec((TRB // 2, 128), lambda i: (i, 0))],
        out_shape=[jax.ShapeDtypeStruct((n, 256), jnp.float32),
                   jax.ShapeDtypeStruct(
                       (pl.cdiv(n, TRB) * (TRB // 2), 128), jnp.float32)],
        compiler_params=pltpu.CompilerParams(
            dimension_semantics=("parallel",)),
    )(u)


def kernel(A_h_index, A_r_index, A_t_index, A_hn_index, A_tn_index, ht1, r1):
    tm, ttail = _transpose_tc(ht1.T)
    partials = _joie_sc(A_h_index.astype(jnp.int32),
                        A_r_index.astype(jnp.int32),
                        A_t_index.astype(jnp.int32),
                        A_hn_index.astype(jnp.int32),
                        A_tn_index.astype(jnp.int32),
                        tm, ttail, r1)
    return jnp.sum(partials) / BATCH
